# baseline (speedup 1.0000x reference)
import functools

import jax
import jax.numpy as jnp
from jax import lax
from jax.experimental import pallas as pl
from jax.experimental.pallas import tpu as pltpu
from jax.experimental.pallas import tpu_sc as plsc

NU = 25000
NI = 25000
DD = 32
NT = NU + 1 + NI
NPAD = 50048
EE = 1600000
BB = 16384

NC = 2
NS = 16
NW = NC * NS
LL = 16

SUB = 128
CHUNK = 384
KSUB = CHUNK // SUB
GA = 198
GB = 66
EP = NS * (GA + GB) * CHUNK
EPA = EP + 2 * CHUNK
SDROWS = EPA // SUB
ROWS_PT = NPAD // NS

_mesh = plsc.VectorSubcoreMesh(
    core_axis_name="c", subcore_axis_name="s", num_cores=NC, num_subcores=NS
)
_sc_params = pltpu.CompilerParams(
    needs_layout_passes=False, use_tc_tiling_on_sc=False
)


@functools.partial(
    pl.kernel,
    out_type=(
        jax.ShapeDtypeStruct((NPAD, DD), jnp.float32),
        jax.ShapeDtypeStruct((NPAD, DD), jnp.float32),
    ),
    mesh=_mesh,
    scratch_types=(
        pltpu.VMEM_SHARED((NPAD, DD), jnp.float32),
        pltpu.VMEM((KSUB, 3, SUB), jnp.int32),
        pltpu.VMEM((KSUB, 3, SUB), jnp.int32),
        pltpu.VMEM((KSUB, 3, SUB), jnp.int32),
        pltpu.VMEM((CHUNK, DD), jnp.float32),
        pltpu.VMEM((CHUNK, DD), jnp.float32),
        pltpu.SemaphoreType.DMA,
        pltpu.SemaphoreType.DMA,
        pltpu.SemaphoreType.DMA,
        pltpu.SemaphoreType.DMA,
        pltpu.SemaphoreType.DMA,
        pltpu.SemaphoreType.DMA,
        pltpu.SemaphoreType.DMA,
    ),
    compiler_params=_sc_params,
)
def _spmm(emb, sd, zrows, out0, out1,
          acc, sdv0, sdv1, sdv2, rows0, rows1,
          sg0, sg1, ss0, ss1, si0, si1, si2):
    cid = lax.axis_index("c")
    sid = lax.axis_index("s")
    wid = cid * NS + sid
    row0 = sid * ROWS_PT
    sdvs = (sdv0, sdv1, sdv2)
    rowss = (rows0, rows1)
    sgs = (sg0, sg1)
    sss = (ss0, ss1)
    sis = (si0, si1, si2)

    pltpu.sync_copy(zrows.at[pl.ds(row0, ROWS_PT)], acc.at[pl.ds(row0, ROWS_PT)])
    plsc.subcore_barrier()

    grp0 = jnp.where(cid == 0, sid * GA, NS * GA + sid * GB)
    niter = jnp.where(cid == 0, GA // 6, GB // 6)

    def idx_fire(g, q):
        roff = (grp0 + g) * KSUB
        pltpu.async_copy(sd.at[pl.ds(roff, KSUB)], sdvs[q], sis[q])

    def idx_drain(q):
        pltpu.make_async_copy(sd.at[pl.ds(0, KSUB)], sdvs[q], sis[q]).wait()

    def gat_fire(q, p):
        for k in range(KSUB):
            pltpu.async_copy(emb.at[sdvs[q].at[k, 0]],
                             rowss[p].at[pl.ds(k * SUB, SUB)], sgs[p])

    def gat_drain(q, p):
        for k in range(KSUB):
            pltpu.make_async_copy(emb.at[sdvs[q].at[k, 0]],
                                  rowss[p].at[pl.ds(k * SUB, SUB)], sgs[p]).wait()

    def sca_fire(q, p):
        for k in range(KSUB):
            pltpu.async_copy(rowss[p].at[pl.ds(k * SUB, SUB)],
                             acc.at[sdvs[q].at[k, 1]], sss[p], add=True)

    def sca_drain(q, p):
        for k in range(KSUB):
            pltpu.make_async_copy(rowss[p].at[pl.ds(k * SUB, SUB)],
                                  acc.at[sdvs[q].at[k, 1]], sss[p]).wait()

    def scale(q, p):
        sdv = sdvs[q]
        rows = rowss[p]

        def mk_blk(k):
            def blk_body(b, c2):
                v16 = plsc.bitcast(sdv[k, 2, pl.ds(b * LL, LL)], jnp.float32)
                for j in range(LL):
                    e = k * SUB + b * LL + j
                    jidx = jnp.broadcast_to(jnp.int32(j), (LL,))
                    vb = jnp.take_along_axis(
                        v16, jidx, axis=0, mode="promise_in_bounds"
                    )
                    r0 = rows[e, pl.ds(0, LL)]
                    r1 = rows[e, pl.ds(LL, LL)]
                    rows[e, pl.ds(0, LL)] = r0 * vb
                    rows[e, pl.ds(LL, LL)] = r1 * vb
                return c2
            return blk_body

        for k in range(KSUB):
            lax.fori_loop(0, SUB // LL, mk_blk(k), 0)

    idx_fire(0, 0)
    idx_fire(1, 1)
    idx_drain(0)
    gat_fire(0, 0)

    def outer(gg, carry):
        g0 = gg * 6
        for j in range(6):
            p = j & 1
            q = j % 3
            g = g0 + j
            gat_drain(q, p)
            idx_drain((j + 1) % 3)
            if j == 0:
                @pl.when(gg > 0)
                def _():
                    sca_drain((j + 2) % 3, 1 - p)
            else:
                sca_drain((j + 2) % 3, 1 - p)
            gat_fire((j + 1) % 3, 1 - p)
            idx_fire(g + 2, (j + 2) % 3)
            scale(q, p)
            sca_fire(q, p)
        return carry

    lax.fori_loop(0, niter, outer, 0)

    gat_drain(0, 0)
    idx_drain(1)
    sca_drain(2, 1)
    plsc.subcore_barrier()

    @pl.when(cid == 0)
    def _():
        pltpu.sync_copy(acc.at[pl.ds(row0, ROWS_PT)], out0.at[pl.ds(row0, ROWS_PT)])

    @pl.when(cid == 1)
    def _():
        pltpu.sync_copy(acc.at[pl.ds(row0, ROWS_PT)], out1.at[pl.ds(row0, ROWS_PT)])


_CROWS = NPAD * DD // 128
_CBLK = 3128


def _combine_body(p0, p1, s_in, e_out, s_out):
    e = p0[...] + p1[...]
    e_out[...] = e
    s_out[...] = s_in[...] + e


def _combine(p0, p1, s_in):
    f = lambda a: a.reshape(_CROWS, 128)
    spec = pl.BlockSpec((_CBLK, 128), lambda i: (i, 0))
    e, s = pl.pallas_call(
        _combine_body,
        grid=(_CROWS // _CBLK,),
        in_specs=[spec, spec, spec],
        out_specs=[spec, spec],
        out_shape=(
            jax.ShapeDtypeStruct((_CROWS, 128), jnp.float32),
            jax.ShapeDtypeStruct((_CROWS, 128), jnp.float32),
        ),
    )(f(p0), f(p1), f(s_in))
    return e.reshape(NPAD, DD), s.reshape(NPAD, DD)


PP = BB // NW
KP = PP // SUB


@functools.partial(
    pl.kernel,
    out_type=jax.ShapeDtypeStruct((BB,), jnp.float32),
    mesh=_mesh,
    scratch_types=(
        pltpu.VMEM((KP, SUB), jnp.int32),
        pltpu.VMEM((KP, SUB), jnp.int32),
        pltpu.VMEM((PP, DD), jnp.float32),
        pltpu.VMEM((PP, DD), jnp.float32),
        pltpu.VMEM((PP,), jnp.float32),
        pltpu.SemaphoreType.DMA,
    ),
    compiler_params=_sc_params,
)
def _gamma(ssum, users2, items2, gout, uidx, iidx, ubuf, vbuf, gloc, sem):
    cid = lax.axis_index("c")
    sid = lax.axis_index("s")
    wid = cid * NS + sid
    r0 = wid * KP

    pltpu.sync_copy(users2.at[pl.ds(r0, KP)], uidx)
    pltpu.sync_copy(items2.at[pl.ds(r0, KP)], iidx)
    for k in range(KP):
        for j in range(SUB // LL):
            v = iidx[k, pl.ds(j * LL, LL)]
            iidx[k, pl.ds(j * LL, LL)] = v + (NU + 1)

    du = [
        pltpu.async_copy(ssum.at[uidx.at[k]], ubuf.at[pl.ds(k * SUB, SUB)], sem)
        for k in range(KP)
    ]
    dv = [
        pltpu.async_copy(ssum.at[iidx.at[k]], vbuf.at[pl.ds(k * SUB, SUB)], sem)
        for k in range(KP)
    ]
    for d in du + dv:
        d.wait()

    iot = jnp.arange(LL, dtype=jnp.int32)

    def grp_body(t, carry):
        pvec = iot + t * LL
        acc = jnp.zeros((LL,), jnp.float32)
        for d in range(DD):
            dvec = jnp.broadcast_to(d, (LL,)).astype(jnp.int32)
            su = plsc.load_gather(ubuf, [pvec, dvec])
            sv = plsc.load_gather(vbuf, [pvec, dvec])
            acc = acc + su * sv
        gloc[pl.ds(t * LL, LL)] = acc * (1.0 / 16.0)
        return carry

    lax.fori_loop(0, PP // LL, grp_body, 0)
    pltpu.sync_copy(gloc, gout.at[pl.ds(wid * PP, PP)])


def _loss_body(g_ref, y_ref, o_ref):
    g = g_ref[...]
    y = y_ref[...]
    l = jnp.maximum(g, 0.0) - g * y + jnp.log1p(jnp.exp(-jnp.abs(g)))
    o_ref[...] = jnp.zeros((8, 128), jnp.float32) + jnp.sum(l) * (1.0 / BB)


def _loss(gamma, y):
    out = pl.pallas_call(
        _loss_body,
        out_shape=jax.ShapeDtypeStruct((8, 128), jnp.float32),
    )(gamma.reshape(128, 128), y.reshape(128, 128))
    return out[0, 0]


def kernel(user_emb, item_emb, edge_index, edge_vals, users, items, labels):
    emb0 = jnp.pad(
        jnp.concatenate([user_emb, item_emb], axis=0), ((0, NPAD - NT), (0, 0))
    )
    dst = edge_index[0]
    src = edge_index[1]
    pad = EPA - EE
    srcm = jnp.pad(src, (0, pad)).reshape(SDROWS, SUB)
    dstm = jnp.pad(dst, (0, pad)).reshape(SDROWS, SUB)
    valm = lax.bitcast_convert_type(
        jnp.pad(edge_vals, (0, pad)), jnp.int32
    ).reshape(SDROWS, SUB)
    sd = jnp.stack([srcm, dstm, valm], axis=1)
    zrows = jnp.zeros((NPAD, DD), jnp.float32)

    e = emb0
    s = emb0
    for _ in range(3):
        p0, p1 = _spmm(e, sd, zrows)
        e, s = _combine(p0, p1, s)

    gamma = _gamma(s, users.reshape(BB // SUB, SUB), items.reshape(BB // SUB, SUB))
    y = labels.astype(jnp.float32)
    return _loss(gamma, y)


# speedup vs baseline: 1.0389x; 1.0389x over previous
import functools

import jax
import jax.numpy as jnp
from jax import lax
from jax.experimental import pallas as pl
from jax.experimental.pallas import tpu as pltpu
from jax.experimental.pallas import tpu_sc as plsc

NU = 25000
NI = 25000
DD = 32
NT = NU + 1 + NI
NPAD = 50048
EE = 1600000
BB = 16384

NC = 2
NS = 16
NW = NC * NS
LL = 16

SUB = 128
CHUNK = 384
KSUB = CHUNK // SUB
GA = 198
GB = 66
EP = NS * (GA + GB) * CHUNK
EPA = EP + 2 * CHUNK
SDROWS = EPA // SUB
ROWS_PT = NPAD // NS

_mesh = plsc.VectorSubcoreMesh(
    core_axis_name="c", subcore_axis_name="s", num_cores=NC, num_subcores=NS
)
_sc_params = pltpu.CompilerParams(
    needs_layout_passes=False, use_tc_tiling_on_sc=False
)


@functools.partial(
    pl.kernel,
    out_type=(
        jax.ShapeDtypeStruct((NPAD, DD), jnp.float32),
        jax.ShapeDtypeStruct((NPAD, DD), jnp.float32),
    ),
    mesh=_mesh,
    scratch_types=(
        pltpu.VMEM_SHARED((NPAD, DD), jnp.float32),
        pltpu.VMEM((KSUB, 2, SUB), jnp.int32),
        pltpu.VMEM((KSUB, 2, SUB), jnp.int32),
        pltpu.VMEM((KSUB, 2, SUB), jnp.int32),
        pltpu.VMEM((CHUNK,), jnp.float32),
        pltpu.VMEM((CHUNK,), jnp.float32),
        pltpu.VMEM((CHUNK,), jnp.float32),
        pltpu.VMEM((CHUNK, DD), jnp.float32),
        pltpu.VMEM((CHUNK, DD), jnp.float32),
        pltpu.SemaphoreType.DMA,
        pltpu.SemaphoreType.DMA,
        pltpu.SemaphoreType.DMA,
        pltpu.SemaphoreType.DMA,
        pltpu.SemaphoreType.DMA,
        pltpu.SemaphoreType.DMA,
        pltpu.SemaphoreType.DMA,
    ),
    compiler_params=_sc_params,
)
def _spmm(emb, sd, valp, zrows, out0, out1,
          acc, sdv0, sdv1, sdv2, valv0, valv1, valv2, rows0, rows1,
          sg0, sg1, ss0, ss1, si0, si1, si2):
    cid = lax.axis_index("c")
    sid = lax.axis_index("s")
    wid = cid * NS + sid
    row0 = sid * ROWS_PT
    sdvs = (sdv0, sdv1, sdv2)
    valvs = (valv0, valv1, valv2)
    rowss = (rows0, rows1)
    sgs = (sg0, sg1)
    sss = (ss0, ss1)
    sis = (si0, si1, si2)

    pltpu.sync_copy(zrows.at[pl.ds(row0, ROWS_PT)], acc.at[pl.ds(row0, ROWS_PT)])
    plsc.subcore_barrier()

    grp0 = jnp.where(cid == 0, sid * GA, NS * GA + sid * GB)
    niter = jnp.where(cid == 0, GA // 6, GB // 6)

    def idx_fire(g, q):
        roff = (grp0 + g) * KSUB
        eoff = (grp0 + g) * CHUNK
        pltpu.async_copy(sd.at[pl.ds(roff, KSUB)], sdvs[q], sis[q])
        pltpu.async_copy(valp.at[pl.ds(eoff, CHUNK)], valvs[q], sis[q])

    def idx_drain(q):
        pltpu.make_async_copy(sd.at[pl.ds(0, KSUB)], sdvs[q], sis[q]).wait()
        pltpu.make_async_copy(valp.at[pl.ds(0, CHUNK)], valvs[q], sis[q]).wait()

    def gat_fire(q, p):
        for k in range(KSUB):
            pltpu.async_copy(emb.at[sdvs[q].at[k, 0]],
                             rowss[p].at[pl.ds(k * SUB, SUB)], sgs[p])

    def gat_drain(q, p):
        for k in range(KSUB):
            pltpu.make_async_copy(emb.at[sdvs[q].at[k, 0]],
                                  rowss[p].at[pl.ds(k * SUB, SUB)], sgs[p]).wait()

    def sca_fire(q, p):
        for k in range(KSUB):
            pltpu.async_copy(rowss[p].at[pl.ds(k * SUB, SUB)],
                             acc.at[sdvs[q].at[k, 1]], sss[p], add=True)

    def sca_drain(q, p):
        for k in range(KSUB):
            pltpu.make_async_copy(rowss[p].at[pl.ds(k * SUB, SUB)],
                                  acc.at[sdvs[q].at[k, 1]], sss[p]).wait()

    def scale(q, p):
        valv = valvs[q]
        rows = rowss[p]

        def blk_body(t, c2):
            v16 = valv[pl.ds(t * LL, LL)]
            for j in range(LL):
                e = t * LL + j
                jidx = jnp.broadcast_to(jnp.int32(j), (LL,))
                vb = jnp.take_along_axis(
                    v16, jidx, axis=0, mode="promise_in_bounds"
                )
                r0 = rows[e, pl.ds(0, LL)]
                r1 = rows[e, pl.ds(LL, LL)]
                rows[e, pl.ds(0, LL)] = r0 * vb
                rows[e, pl.ds(LL, LL)] = r1 * vb
            return c2

        lax.fori_loop(0, CHUNK // LL, blk_body, 0)

    idx_fire(0, 0)
    idx_fire(1, 1)
    idx_drain(0)
    gat_fire(0, 0)

    def outer(gg, carry):
        g0 = gg * 6
        for j in range(6):
            p = j & 1
            q = j % 3
            g = g0 + j
            gat_drain(q, p)
            idx_drain((j + 1) % 3)
            if j == 0:
                @pl.when(gg > 0)
                def _():
                    sca_drain((j + 2) % 3, 1 - p)
            else:
                sca_drain((j + 2) % 3, 1 - p)
            gat_fire((j + 1) % 3, 1 - p)
            idx_fire(g + 2, (j + 2) % 3)
            scale(q, p)
            sca_fire(q, p)
        return carry

    lax.fori_loop(0, niter, outer, 0)

    gat_drain(0, 0)
    idx_drain(1)
    sca_drain(2, 1)
    plsc.subcore_barrier()

    @pl.when(cid == 0)
    def _():
        pltpu.sync_copy(acc.at[pl.ds(row0, ROWS_PT)], out0.at[pl.ds(row0, ROWS_PT)])

    @pl.when(cid == 1)
    def _():
        pltpu.sync_copy(acc.at[pl.ds(row0, ROWS_PT)], out1.at[pl.ds(row0, ROWS_PT)])


_CROWS = NPAD * DD // 128
_CBLK = 3128


def _combine_body(p0, p1, s_in, e_out, s_out):
    e = p0[...] + p1[...]
    e_out[...] = e
    s_out[...] = s_in[...] + e


def _combine(p0, p1, s_in):
    f = lambda a: a.reshape(_CROWS, 128)
    spec = pl.BlockSpec((_CBLK, 128), lambda i: (i, 0))
    e, s = pl.pallas_call(
        _combine_body,
        grid=(_CROWS // _CBLK,),
        in_specs=[spec, spec, spec],
        out_specs=[spec, spec],
        out_shape=(
            jax.ShapeDtypeStruct((_CROWS, 128), jnp.float32),
            jax.ShapeDtypeStruct((_CROWS, 128), jnp.float32),
        ),
    )(f(p0), f(p1), f(s_in))
    return e.reshape(NPAD, DD), s.reshape(NPAD, DD)


PP = BB // NW
KP = PP // SUB


@functools.partial(
    pl.kernel,
    out_type=jax.ShapeDtypeStruct((BB,), jnp.float32),
    mesh=_mesh,
    scratch_types=(
        pltpu.VMEM((KP, SUB), jnp.int32),
        pltpu.VMEM((KP, SUB), jnp.int32),
        pltpu.VMEM((PP, DD), jnp.float32),
        pltpu.VMEM((PP, DD), jnp.float32),
        pltpu.VMEM((PP,), jnp.float32),
        pltpu.SemaphoreType.DMA,
    ),
    compiler_params=_sc_params,
)
def _gamma(ssum, users2, items2, gout, uidx, iidx, ubuf, vbuf, gloc, sem):
    cid = lax.axis_index("c")
    sid = lax.axis_index("s")
    wid = cid * NS + sid
    r0 = wid * KP

    pltpu.sync_copy(users2.at[pl.ds(r0, KP)], uidx)
    pltpu.sync_copy(items2.at[pl.ds(r0, KP)], iidx)
    for k in range(KP):
        for j in range(SUB // LL):
            v = iidx[k, pl.ds(j * LL, LL)]
            iidx[k, pl.ds(j * LL, LL)] = v + (NU + 1)

    du = [
        pltpu.async_copy(ssum.at[uidx.at[k]], ubuf.at[pl.ds(k * SUB, SUB)], sem)
        for k in range(KP)
    ]
    dv = [
        pltpu.async_copy(ssum.at[iidx.at[k]], vbuf.at[pl.ds(k * SUB, SUB)], sem)
        for k in range(KP)
    ]
    for d in du + dv:
        d.wait()

    iot = jnp.arange(LL, dtype=jnp.int32)

    def grp_body(t, carry):
        pvec = iot + t * LL
        acc = jnp.zeros((LL,), jnp.float32)
        for d in range(DD):
            dvec = jnp.broadcast_to(d, (LL,)).astype(jnp.int32)
            su = plsc.load_gather(ubuf, [pvec, dvec])
            sv = plsc.load_gather(vbuf, [pvec, dvec])
            acc = acc + su * sv
        gloc[pl.ds(t * LL, LL)] = acc * (1.0 / 16.0)
        return carry

    lax.fori_loop(0, PP // LL, grp_body, 0)
    pltpu.sync_copy(gloc, gout.at[pl.ds(wid * PP, PP)])


def _loss_body(g_ref, y_ref, o_ref):
    g = g_ref[...]
    y = y_ref[...]
    l = jnp.maximum(g, 0.0) - g * y + jnp.log1p(jnp.exp(-jnp.abs(g)))
    o_ref[...] = jnp.zeros((8, 128), jnp.float32) + jnp.sum(l) * (1.0 / BB)


def _loss(gamma, y):
    out = pl.pallas_call(
        _loss_body,
        out_shape=jax.ShapeDtypeStruct((8, 128), jnp.float32),
    )(gamma.reshape(128, 128), y.reshape(128, 128))
    return out[0, 0]


def kernel(user_emb, item_emb, edge_index, edge_vals, users, items, labels):
    emb0 = jnp.pad(
        jnp.concatenate([user_emb, item_emb], axis=0), ((0, NPAD - NT), (0, 0))
    )
    dst = edge_index[0]
    src = edge_index[1]
    pad = EPA - EE
    srcm = jnp.pad(src, (0, pad)).reshape(SDROWS, SUB)
    dstm = jnp.pad(dst, (0, pad)).reshape(SDROWS, SUB)
    sd = jnp.stack([srcm, dstm], axis=1)
    valp = jnp.pad(edge_vals, (0, pad))
    zrows = jnp.zeros((NPAD, DD), jnp.float32)

    e = emb0
    s = emb0
    for _ in range(3):
        p0, p1 = _spmm(e, sd, valp, zrows)
        e, s = _combine(p0, p1, s)

    gamma = _gamma(s, users.reshape(BB // SUB, SUB), items.reshape(BB // SUB, SUB))
    y = labels.astype(jnp.float32)
    return _loss(gamma, y)
